# Initial kernel scaffold; baseline (speedup 1.0000x reference)
import jax
import jax.numpy as jnp
from jax.experimental import pallas as pl


def kernel(x, embeddings):
    raise NotImplementedError("write your pallas kernel here")


# speedup vs baseline: 1.2650x; 1.2650x over previous
import functools

import jax
import jax.numpy as jnp
from jax import lax
from jax.experimental import pallas as pl
from jax.experimental.pallas import tpu as pltpu
from jax.experimental.pallas import tpu_sc as plsc

N = 18432
D = 64
K = 1024
BLK = 512
GRID = N // BLK

_NC, _NS = 2, 16
NW = _NC * _NS
B_PER_W = N // NW
CH = 96
NCHUNK = B_PER_W // CH


def _tc_body(x_ref, e_ref, idx_ref, avg_ref, loss_ref, perp_ref, loss_acc):
    i = pl.program_id(0)
    xb = x_ref[...]
    e = e_ref[...]
    xx = jnp.sum(xb * xb, axis=-1, keepdims=True)
    ee = jnp.sum(e * e, axis=0, keepdims=True)
    s2 = jnp.dot(2.0 * xb, e, preferred_element_type=jnp.float32)
    dist = (xx + ee) - s2

    idx = jnp.argmin(dist, axis=1).astype(jnp.int32)
    idx_ref[...] = idx

    mind = jnp.min(dist, axis=1)
    one_hot = (lax.broadcasted_iota(jnp.int32, (BLK, K), 1) == idx[:, None])
    counts = jnp.sum(one_hot.astype(jnp.float32), axis=0)[None, :]

    @pl.when(i == 0)
    def _init():
        avg_ref[...] = jnp.zeros_like(avg_ref)
        loss_acc[0, 0] = 0.0

    avg_ref[...] += counts
    loss_acc[0, 0] += jnp.sum(mind)

    @pl.when(i == GRID - 1)
    def _fin():
        avg = avg_ref[...] / float(N)
        avg_ref[...] = avg
        loss_ref[...] = jnp.full((1, 1), loss_acc[0, 0] / float(N * D),
                                 jnp.float32)
        ent = jnp.sum(avg * jnp.log(avg + 1e-10))
        perp_ref[...] = jnp.exp(jnp.full((1, 1), -ent, jnp.float32))


_tc_call = pl.pallas_call(
    _tc_body,
    grid=(GRID,),
    in_specs=[
        pl.BlockSpec((BLK, D), lambda i: (i, 0)),
        pl.BlockSpec((D, K), lambda i: (0, 0)),
    ],
    out_specs=[
        pl.BlockSpec((BLK,), lambda i: (i,)),
        pl.BlockSpec((1, K), lambda i: (0, 0)),
        pl.BlockSpec((1, 1), lambda i: (0, 0)),
        pl.BlockSpec((1, 1), lambda i: (0, 0)),
    ],
    out_shape=[
        jax.ShapeDtypeStruct((N,), jnp.int32),
        jax.ShapeDtypeStruct((1, K), jnp.float32),
        jax.ShapeDtypeStruct((1, 1), jnp.float32),
        jax.ShapeDtypeStruct((1, 1), jnp.float32),
    ],
    scratch_shapes=[pltpu.SMEM((1, 1), jnp.float32)],
)


@functools.cache
def _make_sc_gather():
    @functools.partial(
        pl.kernel,
        mesh=plsc.VectorSubcoreMesh(core_axis_name="c", subcore_axis_name="s"),
        out_type=jax.ShapeDtypeStruct((N, D), jnp.float32),
        scratch_types=[
            pltpu.VMEM((NCHUNK, CH), jnp.int32),
            pltpu.VMEM((B_PER_W, D), jnp.float32),
            pltpu.SemaphoreType.DMA,
        ],
        compiler_params=pltpu.CompilerParams(use_tc_tiling_on_sc=False),
    )
    def _sc_gather(table_hbm, idx_hbm, out_hbm, idx_v, rows_v, sem):
        wid = lax.axis_index("s") * _NC + lax.axis_index("c")
        pltpu.sync_copy(idx_hbm.at[wid], idx_v)
        copies = [
            pltpu.async_copy(table_hbm.at[idx_v.at[c]],
                             rows_v.at[pl.ds(c * CH, CH)], sem)
            for c in range(NCHUNK)
        ]
        for cp in copies:
            cp.wait()
        pltpu.sync_copy(rows_v, out_hbm.at[pl.ds(wid * B_PER_W, B_PER_W)])

    return _sc_gather


def kernel(x, embeddings):
    xf = x.reshape(N, D)
    enc_idx, avg2d, loss2d, perp2d = _tc_call(xf, embeddings)
    table = embeddings.T
    idx3d = enc_idx.reshape(NW, NCHUNK, CH)
    qf = _make_sc_gather()(table, idx3d)
    quantized = qf.reshape(x.shape)
    return (quantized, loss2d.reshape(()), enc_idx,
            avg2d.reshape(K), perp2d.reshape(()))


# speedup vs baseline: 1.3546x; 1.0708x over previous
import functools

import jax
import jax.numpy as jnp
from jax import lax
from jax.experimental import pallas as pl
from jax.experimental.pallas import tpu as pltpu
from jax.experimental.pallas import tpu_sc as plsc

BATCH = 32
SEQ = 576
N = BATCH * SEQ
D = 64
K = 1024
ROWS = 2
BLK = ROWS * SEQ
GRID = BATCH // ROWS

_NC, _NS = 2, 16
NW = _NC * _NS
CH = 96
NCHUNK = SEQ // CH


def _tc_body(x_ref, e_ref, idx_ref, avg_ref, loss_ref, perp_ref, tbl_ref,
             loss_acc):
    i = pl.program_id(0)
    xb = x_ref[...].reshape(BLK, D)
    e = e_ref[...]
    xx = jnp.sum(xb * xb, axis=-1, keepdims=True)
    ee = jnp.sum(e * e, axis=0, keepdims=True)
    s2 = jnp.dot(2.0 * xb, e, preferred_element_type=jnp.float32)
    dist = (xx + ee) - s2

    mind = jnp.min(dist, axis=1)
    eqmask = dist == mind[:, None]
    kiota = lax.broadcasted_iota(jnp.int32, (BLK, K), 1).astype(jnp.float32)
    idx = jnp.min(jnp.where(eqmask, kiota, float(K)), axis=1).astype(jnp.int32)
    idx_ref[pl.ds(i * BLK, BLK)] = idx
    counts = jnp.sum(eqmask.astype(jnp.float32), axis=0)[None, :]

    @pl.when(i == 0)
    def _init():
        avg_ref[...] = jnp.zeros_like(avg_ref)
        loss_acc[0, 0] = 0.0
        tbl_ref[...] = jnp.swapaxes(e, 0, 1)

    avg_ref[...] += counts
    loss_acc[0, 0] += jnp.sum(mind)

    @pl.when(i == GRID - 1)
    def _fin():
        avg = avg_ref[...] / float(N)
        avg_ref[...] = avg
        loss_ref[...] = jnp.full((1, 1), loss_acc[0, 0] / float(N * D),
                                 jnp.float32)
        ent = jnp.sum(avg * jnp.log(avg + 1e-10))
        perp_ref[...] = jnp.exp(jnp.full((1, 1), -ent, jnp.float32))


_tc_call = pl.pallas_call(
    _tc_body,
    grid=(GRID,),
    in_specs=[
        pl.BlockSpec((ROWS, SEQ, D), lambda i: (i, 0, 0)),
        pl.BlockSpec((D, K), lambda i: (0, 0)),
    ],
    out_specs=[
        pl.BlockSpec((N,), lambda i: (0,)),
        pl.BlockSpec((1, K), lambda i: (0, 0)),
        pl.BlockSpec((1, 1), lambda i: (0, 0)),
        pl.BlockSpec((1, 1), lambda i: (0, 0)),
        pl.BlockSpec((K, D), lambda i: (0, 0)),
    ],
    out_shape=[
        jax.ShapeDtypeStruct((N,), jnp.int32),
        jax.ShapeDtypeStruct((1, K), jnp.float32),
        jax.ShapeDtypeStruct((1, 1), jnp.float32),
        jax.ShapeDtypeStruct((1, 1), jnp.float32),
        jax.ShapeDtypeStruct((K, D), jnp.float32),
    ],
    scratch_shapes=[pltpu.SMEM((1, 1), jnp.float32)],
)


@functools.cache
def _make_sc_gather():
    @functools.partial(
        pl.kernel,
        mesh=plsc.VectorSubcoreMesh(core_axis_name="c", subcore_axis_name="s"),
        out_type=jax.ShapeDtypeStruct((BATCH, SEQ, D), jnp.float32),
        scratch_types=[
            pltpu.VMEM((SEQ,), jnp.int32),
            pltpu.VMEM((SEQ, D), jnp.float32),
            pltpu.SemaphoreType.DMA,
        ],
        compiler_params=pltpu.CompilerParams(use_tc_tiling_on_sc=False),
    )
    def _sc_gather(table_hbm, idx_hbm, out_hbm, idx_v, rows_v, sem):
        wid = lax.axis_index("s") * _NC + lax.axis_index("c")
        pltpu.sync_copy(idx_hbm.at[pl.ds(wid * SEQ, SEQ)], idx_v)
        copies = [
            pltpu.async_copy(table_hbm.at[idx_v.at[pl.ds(c * CH, CH)]],
                             rows_v.at[pl.ds(c * CH, CH)], sem)
            for c in range(NCHUNK)
        ]
        for cp in copies:
            cp.wait()
        pltpu.sync_copy(rows_v, out_hbm.at[wid])

    return _sc_gather


def kernel(x, embeddings):
    enc_idx, avg2d, loss2d, perp2d, table = _tc_call(x, embeddings)
    quantized = _make_sc_gather()(table, enc_idx)
    return (quantized, loss2d.reshape(()), enc_idx,
            avg2d.reshape(K), perp2d.reshape(()))
